# baseline (speedup 1.0000x reference)
import functools

import jax
import jax.numpy as jnp
from jax import lax
from jax.experimental import pallas as pl
from jax.experimental.pallas import tpu as pltpu
from jax.experimental.pallas import tpu_sc as plsc

DIM = 16
NW = 32
CB = 8


def _emb_call(B, L):
    b_per_w = B // NW
    n_chunks = b_per_w // CB
    ch_idx = CB * L
    mesh = plsc.VectorSubcoreMesh(core_axis_name="c", subcore_axis_name="s")

    @functools.partial(
        pl.kernel,
        mesh=mesh,
        out_type=jax.ShapeDtypeStruct((B, L, DIM), jnp.float32),
        scratch_types=[
            pltpu.VMEM((ch_idx,), jnp.int32),
            pltpu.VMEM((2, CB, L, DIM), jnp.float32),
            pltpu.SemaphoreType.DMA,
            pltpu.SemaphoreType.DMA,
            pltpu.SemaphoreType.DMA,
        ],
        compiler_params=pltpu.CompilerParams(use_tc_tiling_on_sc=False),
    )
    def emb(idx_hbm, table_hbm, out_hbm, idxbuf, rowbuf, gsem, osem0, osem1):
        wid = lax.axis_index("s") * 2 + lax.axis_index("c")
        base_b = wid * b_per_w
        osems = (osem0, osem1)

        def half_body(ci, b):
            b0 = base_b + ci * CB
            pltpu.sync_copy(idx_hbm.at[pl.ds(b0 * L, ch_idx)], idxbuf)

            def add_body(i, c):
                for s in range(4):
                    sl = pl.ds(i * 64 + s * 16, 16)
                    idxbuf[sl] = idxbuf[sl] + 1
                return c

            lax.fori_loop(0, ch_idx // 64, add_body, 0)

            @pl.when(ci >= 2)
            def _():
                pltpu.make_async_copy(
                    rowbuf.at[b], out_hbm.at[pl.ds(b0 - 2 * CB, CB)], osems[b]
                ).wait()

            copies = []
            for j in range(CB):
                dst = rowbuf.at[b].at[j]
                copies.append(pltpu.async_copy(
                    table_hbm.at[idxbuf.at[pl.ds(j * L, 128)]],
                    dst.at[pl.ds(0, 128)], gsem))
                copies.append(pltpu.async_copy(
                    table_hbm.at[idxbuf.at[pl.ds(j * L + 128, L - 128)]],
                    dst.at[pl.ds(128, L - 128)], gsem))
            for c in copies:
                c.wait()

            pltpu.async_copy(
                rowbuf.at[b], out_hbm.at[pl.ds(b0, CB)], osems[b]
            )
            return b

        def chunk_pair(g, carry):
            half_body(2 * g, 0)
            half_body(2 * g + 1, 1)
            return carry

        lax.fori_loop(0, n_chunks // 2, chunk_pair, 0)

        for b in range(2):
            b0 = base_b + (n_chunks - 2 + b) * CB
            pltpu.make_async_copy(
                rowbuf.at[b], out_hbm.at[pl.ds(b0, CB)], osems[b]
            ).wait()

    return emb


def kernel(action, table):
    B, L, _ = action.shape
    idx = action.reshape(B * L)
    return _emb_call(B, L)(idx, table)
